# Initial kernel scaffold; baseline (speedup 1.0000x reference)
import jax
import jax.numpy as jnp
from jax.experimental import pallas as pl


def kernel(fluidPosition, fluidVolume, fluidDistances, fluidRadialDistances, fluidDensity, i, j):
    raise NotImplementedError("write your pallas kernel here")


# speedup vs baseline: 163.9265x; 163.9265x over previous
import functools

import jax
import jax.numpy as jnp
import numpy as np
from jax import lax
from jax.experimental import pallas as pl
from jax.experimental.pallas import tpu as pltpu
from jax.experimental.pallas import tpu_sc as plsc

N = 100000
E = 3200000
SUPPORT = 0.05
DELTA = 0.1
C0 = float(10.0 * np.sqrt(2.0 * 9.81 * 0.3))
EPS = SUPPORT * SUPPORT * 0.1
REST_DENSITY = 1000.0
KGRAD_C = float(7.0 / (np.pi * SUPPORT * SUPPORT))
SCALE = float(2.0 * SUPPORT * DELTA * C0)
EPS_LIM = 0.0001 * SUPPORT

NW = 32
EP = 3276800
RROWS = EP // 128
RW = RROWS // NW
KB = 16
NCH = RW // KB
NP = 100352
NT = NP // 16
NR, NL = 196, 512

_MESH = dict(core_axis_name="c", subcore_axis_name="s", num_cores=2,
             num_subcores=16)

_EDGE_BUFS = [
    pltpu.VMEM((KB, 128), jnp.int32),
    pltpu.VMEM((KB, 128), jnp.int32),
    pltpu.VMEM((KB, 128), jnp.float32),
    pltpu.VMEM((KB, 128), jnp.float32),
    pltpu.VMEM((KB, 128), jnp.float32),
]


def _gradw(dxv, dyv, radv):
  q = jnp.clip(radv, 0.0, 1.0)
  omq = 1.0 - q
  f = (-20.0 * KGRAD_C / SUPPORT) * q * omq * omq * omq
  return f * dxv, f * dyv


def _load_edges(refs, bufs, rb):
  rs = pl.ds(rb, KB)
  for r, bbuf in zip(refs, bufs):
    pltpu.sync_copy(r.at[rs], bbuf)


def _zero_accs(z_hbm, tmp, accs, s):
  ns = pl.ds(s * NT, NT)
  for a in accs:
    pltpu.sync_copy(z_hbm.at[ns], tmp)
    pltpu.sync_copy(tmp, a.at[ns])
  plsc.subcore_barrier()


def _write_out(accs, tmp, out_hbm, c, s):
  plsc.subcore_barrier()
  ns = pl.ds(s * NT, NT)
  for k, a in enumerate(accs):
    pltpu.sync_copy(a.at[ns], tmp)
    pltpu.sync_copy(tmp, out_hbm.at[c, k, ns])


def _sc_pass_a(iE, jE, dxE, dyE, radE, volP, zN):
  mesh = plsc.VectorSubcoreMesh(**_MESH)

  @functools.partial(
      pl.kernel,
      out_type=jax.ShapeDtypeStruct((2, 4, NP), jnp.float32),
      mesh=mesh,
      scratch_types=[
          *[pltpu.VMEM_SHARED((NP,), jnp.float32) for _ in range(4)],
          *_EDGE_BUFS,
          pltpu.VMEM((KB, 128), jnp.float32),
          *[pltpu.VMEM((KB, 128), jnp.float32) for _ in range(4)],
          pltpu.VMEM((NT,), jnp.float32),
          pltpu.SemaphoreType.DMA,
          pltpu.SemaphoreType.DMA,
      ],
  )
  def body(i_hbm, j_hbm, dx_hbm, dy_hbm, rad_hbm, vol_hbm, z_hbm, out_hbm,
           a0, a1, a2, a3, bi, bj, bdx, bdy, brad, bvol,
           v0, v1, v2, v3, tmp, gsem, ssem):
    c = lax.axis_index("c")
    s = lax.axis_index("s")
    wid = s * 2 + c
    accs = (a0, a1, a2, a3)
    vals = (v0, v1, v2, v3)
    _zero_accs(z_hbm, tmp, accs, s)

    @pl.loop(0, NCH)
    def _chunk(t):
      _load_edges((i_hbm, j_hbm, dx_hbm, dy_hbm, rad_hbm),
                  (bi, bj, bdx, bdy, brad), wid * RW + t * KB)
      gds = [pltpu.async_copy(vol_hbm.at[bj.at[b]], bvol.at[b], gsem)
             for b in range(KB)]
      for d in gds:
        d.wait()

      @pl.loop(0, KB)
      def _row(b):
        for l in range(8):
          sl = pl.ds(l * 16, 16)
          dxv = bdx[b, sl]
          dyv = bdy[b, sl]
          radv = brad[b, sl]
          gwx, gwy = _gradw(dxv, dyv, radv)
          rbx = -dxv * radv * SUPPORT
          rby = -dyv * radv * SUPPORT
          fac = bvol[b, sl] * 2.0
          v0[b, sl] = rbx * gwx * fac
          v1[b, sl] = rbx * gwy * fac
          v2[b, sl] = rby * gwx * fac
          v3[b, sl] = rby * gwy * fac

      sds = [pltpu.async_copy(vals[k].at[b], accs[k].at[bi.at[b]], ssem,
                              add=True)
             for b in range(KB) for k in range(4)]
      for d in sds:
        d.wait()

    _write_out(accs, tmp, out_hbm, c, s)

  return body(iE, jE, dxE, dyE, radE, volP, zN)


def _sc_pass_c(iE, jE, dxE, dyE, radE, li0P, li1P, li2P, li3P, rhoP, volP,
               zN):
  mesh = plsc.VectorSubcoreMesh(**_MESH)

  @functools.partial(
      pl.kernel,
      out_type=jax.ShapeDtypeStruct((2, 2, NP), jnp.float32),
      mesh=mesh,
      scratch_types=[
          *[pltpu.VMEM_SHARED((NP,), jnp.float32) for _ in range(2)],
          *_EDGE_BUFS,
          *[pltpu.VMEM((KB, 128), jnp.float32) for _ in range(7)],
          *[pltpu.VMEM((KB, 128), jnp.float32) for _ in range(2)],
          pltpu.VMEM((NT,), jnp.float32),
          pltpu.SemaphoreType.DMA,
          pltpu.SemaphoreType.DMA,
      ],
  )
  def body(i_hbm, j_hbm, dx_hbm, dy_hbm, rad_hbm, l0_hbm, l1_hbm, l2_hbm,
           l3_hbm, rho_hbm, vol_hbm, z_hbm, out_hbm,
           ax, ay, bi, bj, bdx, bdy, brad,
           bl0, bl1, bl2, bl3, brhoi, brhoj, bvolj,
           vx, vy, tmp, gsem, ssem):
    c = lax.axis_index("c")
    s = lax.axis_index("s")
    wid = s * 2 + c
    accs = (ax, ay)
    _zero_accs(z_hbm, tmp, accs, s)

    @pl.loop(0, NCH)
    def _chunk(t):
      _load_edges((i_hbm, j_hbm, dx_hbm, dy_hbm, rad_hbm),
                  (bi, bj, bdx, bdy, brad), wid * RW + t * KB)
      gds = []
      for b in range(KB):
        ii = bi.at[b]
        jj = bj.at[b]
        gds.append(pltpu.async_copy(l0_hbm.at[ii], bl0.at[b], gsem))
        gds.append(pltpu.async_copy(l1_hbm.at[ii], bl1.at[b], gsem))
        gds.append(pltpu.async_copy(l2_hbm.at[ii], bl2.at[b], gsem))
        gds.append(pltpu.async_copy(l3_hbm.at[ii], bl3.at[b], gsem))
        gds.append(pltpu.async_copy(rho_hbm.at[ii], brhoi.at[b], gsem))
        gds.append(pltpu.async_copy(rho_hbm.at[jj], brhoj.at[b], gsem))
        gds.append(pltpu.async_copy(vol_hbm.at[jj], bvolj.at[b], gsem))
      for d in gds:
        d.wait()

      @pl.loop(0, KB)
      def _row(b):
        for l in range(8):
          sl = pl.ds(l * 16, 16)
          dxv = bdx[b, sl]
          dyv = bdy[b, sl]
          radv = brad[b, sl]
          gwx, gwy = _gradw(dxv, dyv, radv)
          ngx = bl0[b, sl] * gwx + bl1[b, sl] * gwy
          ngy = bl2[b, sl] * gwx + bl3[b, sl] * gwy
          dwij_mag = jnp.abs(gwx) + jnp.abs(gwy)
          norm_mag = jnp.abs(ngx) + jnp.abs(ngy)
          change = jnp.abs(norm_mag - dwij_mag) / (dwij_mag + EPS_LIM)
          sel = change < 0.1
          gx = jnp.where(sel, ngx, gwx)
          gy = jnp.where(sel, ngy, gwy)
          fac2 = (brhoj[b, sl] - brhoi[b, sl]) * bvolj[b, sl] * 2.0
          vx[b, sl] = fac2 * gx
          vy[b, sl] = fac2 * gy

      sds = []
      for b in range(KB):
        ii = bi.at[b]
        sds.append(pltpu.async_copy(vx.at[b], ax.at[ii], ssem, add=True))
        sds.append(pltpu.async_copy(vy.at[b], ay.at[ii], ssem, add=True))
      for d in sds:
        d.wait()

    _write_out(accs, tmp, out_hbm, c, s)

  return body(iE, jE, dxE, dyE, radE, li0P, li1P, li2P, li3P, rhoP, volP, zN)


def _sc_pass_e(iE, jE, dxE, dyE, radE, gxP, gyP, rhoP, volP, zN):
  mesh = plsc.VectorSubcoreMesh(**_MESH)

  @functools.partial(
      pl.kernel,
      out_type=jax.ShapeDtypeStruct((2, 1, NP), jnp.float32),
      mesh=mesh,
      scratch_types=[
          pltpu.VMEM_SHARED((NP,), jnp.float32),
          *_EDGE_BUFS,
          *[pltpu.VMEM((KB, 128), jnp.float32) for _ in range(7)],
          pltpu.VMEM((KB, 128), jnp.float32),
          pltpu.VMEM((NT,), jnp.float32),
          pltpu.SemaphoreType.DMA,
          pltpu.SemaphoreType.DMA,
      ],
  )
  def body(i_hbm, j_hbm, dx_hbm, dy_hbm, rad_hbm, gx_hbm, gy_hbm, rho_hbm,
           vol_hbm, z_hbm, out_hbm,
           acc, bi, bj, bdx, bdy, brad,
           bgxi, bgyi, brhoi, bgxj, bgyj, brhoj, bvolj,
           vv, tmp, gsem, ssem):
    c = lax.axis_index("c")
    s = lax.axis_index("s")
    wid = s * 2 + c
    _zero_accs(z_hbm, tmp, (acc,), s)

    @pl.loop(0, NCH)
    def _chunk(t):
      _load_edges((i_hbm, j_hbm, dx_hbm, dy_hbm, rad_hbm),
                  (bi, bj, bdx, bdy, brad), wid * RW + t * KB)
      gds = []
      for b in range(KB):
        ii = bi.at[b]
        jj = bj.at[b]
        gds.append(pltpu.async_copy(gx_hbm.at[ii], bgxi.at[b], gsem))
        gds.append(pltpu.async_copy(gy_hbm.at[ii], bgyi.at[b], gsem))
        gds.append(pltpu.async_copy(rho_hbm.at[ii], brhoi.at[b], gsem))
        gds.append(pltpu.async_copy(gx_hbm.at[jj], bgxj.at[b], gsem))
        gds.append(pltpu.async_copy(gy_hbm.at[jj], bgyj.at[b], gsem))
        gds.append(pltpu.async_copy(rho_hbm.at[jj], brhoj.at[b], gsem))
        gds.append(pltpu.async_copy(vol_hbm.at[jj], bvolj.at[b], gsem))
      for d in gds:
        d.wait()

      @pl.loop(0, KB)
      def _row(b):
        for l in range(8):
          sl = pl.ds(l * 16, 16)
          dxv = bdx[b, sl]
          dyv = bdy[b, sl]
          radv = brad[b, sl]
          gwx, gwy = _gradw(dxv, dyv, radv)
          rbx = -dxv * radv * SUPPORT
          rby = -dyv * radv * SUPPORT
          rji2 = rbx * rbx + rby * rby + EPS
          density_term = 0.5 * ((bgxi[b, sl] + bgxj[b, sl]) * rbx +
                                (bgyi[b, sl] + bgyj[b, sl]) * rby)
          diffusion_term = brhoj[b, sl] - brhoi[b, sl]
          grad_term = (gwx * rbx + gwy * rby) / rji2
          prod = (diffusion_term + density_term) * grad_term
          vv[b, sl] = prod * bvolj[b, sl]

      sds = [pltpu.async_copy(vv.at[b], acc.at[bi.at[b]], ssem, add=True)
             for b in range(KB)]
      for d in sds:
        d.wait()

    _write_out((acc,), tmp, out_hbm, c, s)

  return body(iE, jE, dxE, dyE, radE, gxP, gyP, rhoP, volP, zN)


def _tc_pinv_body(m_ref, dens, li0, li1, li2, li3, rho):
  a = m_ref[0, 0] + m_ref[1, 0]
  b = m_ref[0, 1] + m_ref[1, 1]
  c = m_ref[0, 2] + m_ref[1, 2]
  d = m_ref[0, 3] + m_ref[1, 3]
  det = a * d - b * c
  frob2 = a * a + b * b + c * c + d * d
  use = jnp.abs(det) > 1e-6 * frob2
  sdet = jnp.where(use, det, 1.0)
  sfro = jnp.maximum(frob2, 1e-30)
  li0[...] = jnp.where(use, d / sdet, a / sfro)
  li1[...] = jnp.where(use, -b / sdet, c / sfro)
  li2[...] = jnp.where(use, -c / sdet, b / sfro)
  li3[...] = jnp.where(use, a / sdet, d / sfro)
  rho[...] = dens[...] * REST_DENSITY


def _tc_combine2_body(g_ref, gx, gy):
  gx[...] = g_ref[0, 0] + g_ref[1, 0]
  gy[...] = g_ref[0, 1] + g_ref[1, 1]


def _tc_final_body(dp, out):
  out[...] = SCALE * (dp[0, 0] + dp[1, 0])


def kernel(fluidPosition, fluidVolume, fluidDistances, fluidRadialDistances,
           fluidDensity, i, j):
  del fluidPosition
  i = i.astype(jnp.int32)
  j = j.astype(jnp.int32)
  pad = EP - E
  iE = jnp.pad(i, (0, pad)).reshape(RROWS, 128)
  jE = jnp.pad(j, (0, pad)).reshape(RROWS, 128)
  dxE = jnp.pad(fluidDistances[:, 0], (0, pad)).reshape(RROWS, 128)
  dyE = jnp.pad(fluidDistances[:, 1], (0, pad)).reshape(RROWS, 128)
  radE = jnp.pad(fluidRadialDistances, (0, pad)).reshape(RROWS, 128)
  volP = jnp.pad(fluidVolume, (0, NP - N))
  densP = jnp.pad(fluidDensity, (0, NP - N))
  zN = jnp.zeros((NP,), jnp.float32)
  f32 = jnp.float32
  shp = jax.ShapeDtypeStruct((NR, NL), f32)

  Mpart = _sc_pass_a(iE, jE, dxE, dyE, radE, volP, zN)

  li0, li1, li2, li3, rho2 = pl.pallas_call(
      _tc_pinv_body, out_shape=[shp] * 5)(
          Mpart.reshape(2, 4, NR, NL), densP.reshape(NR, NL))

  Gpart = _sc_pass_c(iE, jE, dxE, dyE, radE,
                     li0.reshape(NP), li1.reshape(NP), li2.reshape(NP),
                     li3.reshape(NP), rho2.reshape(NP), volP, zN)

  gx2, gy2 = pl.pallas_call(_tc_combine2_body, out_shape=[shp] * 2)(
      Gpart.reshape(2, 2, NR, NL))

  Dpart = _sc_pass_e(iE, jE, dxE, dyE, radE,
                     gx2.reshape(NP), gy2.reshape(NP), rho2.reshape(NP),
                     volP, zN)

  out2 = pl.pallas_call(_tc_final_body, out_shape=shp)(
      Dpart.reshape(2, 1, NR, NL))
  return out2.reshape(NP)[:N]


# speedup vs baseline: 309.8524x; 1.8902x over previous
import functools

import jax
import jax.numpy as jnp
import numpy as np
from jax import lax
from jax.experimental import pallas as pl
from jax.experimental.pallas import tpu as pltpu
from jax.experimental.pallas import tpu_sc as plsc

N = 100000
E = 3200000
SUPPORT = 0.05
DELTA = 0.1
C0 = float(10.0 * np.sqrt(2.0 * 9.81 * 0.3))
EPS = SUPPORT * SUPPORT * 0.1
REST_DENSITY = 1000.0
KGRAD_C = float(7.0 / (np.pi * SUPPORT * SUPPORT))
SCALE = float(2.0 * SUPPORT * DELTA * C0)
EPS_LIM = 0.0001 * SUPPORT

NW = 32
EP = 3276800
RROWS = EP // 128
RW = RROWS // NW
KB = 16
NCH = RW // KB
NP = 100352
NT = NP // 16
NR, NL = 196, 512

_MESH = dict(core_axis_name="c", subcore_axis_name="s", num_cores=2,
             num_subcores=16)

_EDGE_BUFS = [
    pltpu.VMEM((KB, 128), jnp.int32),
    pltpu.VMEM((KB, 128), jnp.int32),
    pltpu.VMEM((KB, 128), jnp.float32),
    pltpu.VMEM((KB, 128), jnp.float32),
    pltpu.VMEM((KB, 128), jnp.float32),
]


def _gradw(dxv, dyv, radv):
  q = jnp.clip(radv, 0.0, 1.0)
  omq = 1.0 - q
  f = (-20.0 * KGRAD_C / SUPPORT) * q * omq * omq * omq
  return f * dxv, f * dyv


def _load_edges(refs, bufs, rb):
  rs = pl.ds(rb, KB)
  for r, bbuf in zip(refs, bufs):
    pltpu.sync_copy(r.at[rs], bbuf)


def _zero_accs(z_hbm, tmp, accs, s):
  ns = pl.ds(s * NT, NT)
  for a in accs:
    pltpu.sync_copy(z_hbm.at[ns], tmp)
    pltpu.sync_copy(tmp, a.at[ns])


def _stage_tables(pairs, tmp, s):
  ns = pl.ds(s * NT, NT)
  for hbm, sp in pairs:
    pltpu.sync_copy(hbm.at[ns], tmp)
    pltpu.sync_copy(tmp, sp.at[ns])


def _write_out(accs, tmp, out_hbm, c, s):
  plsc.subcore_barrier()
  ns = pl.ds(s * NT, NT)
  for k, a in enumerate(accs):
    pltpu.sync_copy(a.at[ns], tmp)
    pltpu.sync_copy(tmp, out_hbm.at[c, k, ns])


def _sc_pass_a(iE, jE, dxE, dyE, radE, volP, zN):
  mesh = plsc.VectorSubcoreMesh(**_MESH)

  @functools.partial(
      pl.kernel,
      out_type=jax.ShapeDtypeStruct((2, 4, NP), jnp.float32),
      mesh=mesh,
      scratch_types=[
          *[pltpu.VMEM_SHARED((NP,), jnp.float32) for _ in range(4)],
          pltpu.VMEM_SHARED((NP,), jnp.float32),
          *_EDGE_BUFS,
          pltpu.VMEM((KB, 128), jnp.float32),
          *[pltpu.VMEM((KB, 128), jnp.float32) for _ in range(4)],
          pltpu.VMEM((NT,), jnp.float32),
          pltpu.SemaphoreType.DMA,
          pltpu.SemaphoreType.DMA,
      ],
  )
  def body(i_hbm, j_hbm, dx_hbm, dy_hbm, rad_hbm, vol_hbm, z_hbm, out_hbm,
           a0, a1, a2, a3, vol_sp, bi, bj, bdx, bdy, brad, bvol,
           v0, v1, v2, v3, tmp, gsem, ssem):
    c = lax.axis_index("c")
    s = lax.axis_index("s")
    wid = s * 2 + c
    accs = (a0, a1, a2, a3)
    vals = (v0, v1, v2, v3)
    _zero_accs(z_hbm, tmp, accs, s)
    _stage_tables([(vol_hbm, vol_sp)], tmp, s)
    plsc.subcore_barrier()

    @pl.loop(0, NCH)
    def _chunk(t):
      _load_edges((i_hbm, j_hbm, dx_hbm, dy_hbm, rad_hbm),
                  (bi, bj, bdx, bdy, brad), wid * RW + t * KB)
      gds = [pltpu.async_copy(vol_sp.at[bj.at[b]], bvol.at[b], gsem)
             for b in range(KB)]
      for d in gds:
        d.wait()

      @pl.loop(0, KB)
      def _row(b):
        for l in range(8):
          sl = pl.ds(l * 16, 16)
          dxv = bdx[b, sl]
          dyv = bdy[b, sl]
          radv = brad[b, sl]
          gwx, gwy = _gradw(dxv, dyv, radv)
          rbx = -dxv * radv * SUPPORT
          rby = -dyv * radv * SUPPORT
          fac = bvol[b, sl] * 2.0
          v0[b, sl] = rbx * gwx * fac
          v1[b, sl] = rbx * gwy * fac
          v2[b, sl] = rby * gwx * fac
          v3[b, sl] = rby * gwy * fac

      sds = [pltpu.async_copy(vals[k].at[b], accs[k].at[bi.at[b]], ssem,
                              add=True)
             for b in range(KB) for k in range(4)]
      for d in sds:
        d.wait()

    _write_out(accs, tmp, out_hbm, c, s)

  return body(iE, jE, dxE, dyE, radE, volP, zN)


def _sc_pass_c(iE, jE, dxE, dyE, radE, li0P, li1P, li2P, li3P, rhoP, volP,
               zN):
  mesh = plsc.VectorSubcoreMesh(**_MESH)

  @functools.partial(
      pl.kernel,
      out_type=jax.ShapeDtypeStruct((2, 2, NP), jnp.float32),
      mesh=mesh,
      scratch_types=[
          *[pltpu.VMEM_SHARED((NP,), jnp.float32) for _ in range(2)],
          *[pltpu.VMEM_SHARED((NP,), jnp.float32) for _ in range(6)],
          *_EDGE_BUFS,
          *[pltpu.VMEM((KB, 128), jnp.float32) for _ in range(7)],
          *[pltpu.VMEM((KB, 128), jnp.float32) for _ in range(2)],
          pltpu.VMEM((NT,), jnp.float32),
          pltpu.SemaphoreType.DMA,
          pltpu.SemaphoreType.DMA,
      ],
  )
  def body(i_hbm, j_hbm, dx_hbm, dy_hbm, rad_hbm, l0_hbm, l1_hbm, l2_hbm,
           l3_hbm, rho_hbm, vol_hbm, z_hbm, out_hbm,
           ax, ay, l0_sp, l1_sp, l2_sp, l3_sp, rho_sp, vol_sp,
           bi, bj, bdx, bdy, brad,
           bl0, bl1, bl2, bl3, brhoi, brhoj, bvolj,
           vx, vy, tmp, gsem, ssem):
    c = lax.axis_index("c")
    s = lax.axis_index("s")
    wid = s * 2 + c
    accs = (ax, ay)
    _zero_accs(z_hbm, tmp, accs, s)
    _stage_tables([(l0_hbm, l0_sp), (l1_hbm, l1_sp), (l2_hbm, l2_sp),
                   (l3_hbm, l3_sp), (rho_hbm, rho_sp), (vol_hbm, vol_sp)],
                  tmp, s)
    plsc.subcore_barrier()

    @pl.loop(0, NCH)
    def _chunk(t):
      _load_edges((i_hbm, j_hbm, dx_hbm, dy_hbm, rad_hbm),
                  (bi, bj, bdx, bdy, brad), wid * RW + t * KB)
      gds = []
      for b in range(KB):
        ii = bi.at[b]
        jj = bj.at[b]
        gds.append(pltpu.async_copy(l0_sp.at[ii], bl0.at[b], gsem))
        gds.append(pltpu.async_copy(l1_sp.at[ii], bl1.at[b], gsem))
        gds.append(pltpu.async_copy(l2_sp.at[ii], bl2.at[b], gsem))
        gds.append(pltpu.async_copy(l3_sp.at[ii], bl3.at[b], gsem))
        gds.append(pltpu.async_copy(rho_sp.at[ii], brhoi.at[b], gsem))
        gds.append(pltpu.async_copy(rho_sp.at[jj], brhoj.at[b], gsem))
        gds.append(pltpu.async_copy(vol_sp.at[jj], bvolj.at[b], gsem))
      for d in gds:
        d.wait()

      @pl.loop(0, KB)
      def _row(b):
        for l in range(8):
          sl = pl.ds(l * 16, 16)
          dxv = bdx[b, sl]
          dyv = bdy[b, sl]
          radv = brad[b, sl]
          gwx, gwy = _gradw(dxv, dyv, radv)
          ngx = bl0[b, sl] * gwx + bl1[b, sl] * gwy
          ngy = bl2[b, sl] * gwx + bl3[b, sl] * gwy
          dwij_mag = jnp.abs(gwx) + jnp.abs(gwy)
          norm_mag = jnp.abs(ngx) + jnp.abs(ngy)
          change = jnp.abs(norm_mag - dwij_mag) / (dwij_mag + EPS_LIM)
          sel = change < 0.1
          gx = jnp.where(sel, ngx, gwx)
          gy = jnp.where(sel, ngy, gwy)
          fac2 = (brhoj[b, sl] - brhoi[b, sl]) * bvolj[b, sl] * 2.0
          vx[b, sl] = fac2 * gx
          vy[b, sl] = fac2 * gy

      sds = []
      for b in range(KB):
        ii = bi.at[b]
        sds.append(pltpu.async_copy(vx.at[b], ax.at[ii], ssem, add=True))
        sds.append(pltpu.async_copy(vy.at[b], ay.at[ii], ssem, add=True))
      for d in sds:
        d.wait()

    _write_out(accs, tmp, out_hbm, c, s)

  return body(iE, jE, dxE, dyE, radE, li0P, li1P, li2P, li3P, rhoP, volP, zN)


def _sc_pass_e(iE, jE, dxE, dyE, radE, gxP, gyP, rhoP, volP, zN):
  mesh = plsc.VectorSubcoreMesh(**_MESH)

  @functools.partial(
      pl.kernel,
      out_type=jax.ShapeDtypeStruct((2, 1, NP), jnp.float32),
      mesh=mesh,
      scratch_types=[
          pltpu.VMEM_SHARED((NP,), jnp.float32),
          *[pltpu.VMEM_SHARED((NP,), jnp.float32) for _ in range(4)],
          *_EDGE_BUFS,
          *[pltpu.VMEM((KB, 128), jnp.float32) for _ in range(7)],
          pltpu.VMEM((KB, 128), jnp.float32),
          pltpu.VMEM((NT,), jnp.float32),
          pltpu.SemaphoreType.DMA,
          pltpu.SemaphoreType.DMA,
      ],
  )
  def body(i_hbm, j_hbm, dx_hbm, dy_hbm, rad_hbm, gx_hbm, gy_hbm, rho_hbm,
           vol_hbm, z_hbm, out_hbm,
           acc, gx_sp, gy_sp, rho_sp, vol_sp, bi, bj, bdx, bdy, brad,
           bgxi, bgyi, brhoi, bgxj, bgyj, brhoj, bvolj,
           vv, tmp, gsem, ssem):
    c = lax.axis_index("c")
    s = lax.axis_index("s")
    wid = s * 2 + c
    _zero_accs(z_hbm, tmp, (acc,), s)
    _stage_tables([(gx_hbm, gx_sp), (gy_hbm, gy_sp), (rho_hbm, rho_sp),
                   (vol_hbm, vol_sp)], tmp, s)
    plsc.subcore_barrier()

    @pl.loop(0, NCH)
    def _chunk(t):
      _load_edges((i_hbm, j_hbm, dx_hbm, dy_hbm, rad_hbm),
                  (bi, bj, bdx, bdy, brad), wid * RW + t * KB)
      gds = []
      for b in range(KB):
        ii = bi.at[b]
        jj = bj.at[b]
        gds.append(pltpu.async_copy(gx_sp.at[ii], bgxi.at[b], gsem))
        gds.append(pltpu.async_copy(gy_sp.at[ii], bgyi.at[b], gsem))
        gds.append(pltpu.async_copy(rho_sp.at[ii], brhoi.at[b], gsem))
        gds.append(pltpu.async_copy(gx_sp.at[jj], bgxj.at[b], gsem))
        gds.append(pltpu.async_copy(gy_sp.at[jj], bgyj.at[b], gsem))
        gds.append(pltpu.async_copy(rho_sp.at[jj], brhoj.at[b], gsem))
        gds.append(pltpu.async_copy(vol_sp.at[jj], bvolj.at[b], gsem))
      for d in gds:
        d.wait()

      @pl.loop(0, KB)
      def _row(b):
        for l in range(8):
          sl = pl.ds(l * 16, 16)
          dxv = bdx[b, sl]
          dyv = bdy[b, sl]
          radv = brad[b, sl]
          gwx, gwy = _gradw(dxv, dyv, radv)
          rbx = -dxv * radv * SUPPORT
          rby = -dyv * radv * SUPPORT
          rji2 = rbx * rbx + rby * rby + EPS
          density_term = 0.5 * ((bgxi[b, sl] + bgxj[b, sl]) * rbx +
                                (bgyi[b, sl] + bgyj[b, sl]) * rby)
          diffusion_term = brhoj[b, sl] - brhoi[b, sl]
          grad_term = (gwx * rbx + gwy * rby) / rji2
          prod = (diffusion_term + density_term) * grad_term
          vv[b, sl] = prod * bvolj[b, sl]

      sds = [pltpu.async_copy(vv.at[b], acc.at[bi.at[b]], ssem, add=True)
             for b in range(KB)]
      for d in sds:
        d.wait()

    _write_out((acc,), tmp, out_hbm, c, s)

  return body(iE, jE, dxE, dyE, radE, gxP, gyP, rhoP, volP, zN)


def _tc_pinv_body(m_ref, dens, li0, li1, li2, li3, rho):
  a = m_ref[0, 0] + m_ref[1, 0]
  b = m_ref[0, 1] + m_ref[1, 1]
  c = m_ref[0, 2] + m_ref[1, 2]
  d = m_ref[0, 3] + m_ref[1, 3]
  det = a * d - b * c
  frob2 = a * a + b * b + c * c + d * d
  use = jnp.abs(det) > 1e-6 * frob2
  sdet = jnp.where(use, det, 1.0)
  sfro = jnp.maximum(frob2, 1e-30)
  li0[...] = jnp.where(use, d / sdet, a / sfro)
  li1[...] = jnp.where(use, -b / sdet, c / sfro)
  li2[...] = jnp.where(use, -c / sdet, b / sfro)
  li3[...] = jnp.where(use, a / sdet, d / sfro)
  rho[...] = dens[...] * REST_DENSITY


def _tc_combine2_body(g_ref, gx, gy):
  gx[...] = g_ref[0, 0] + g_ref[1, 0]
  gy[...] = g_ref[0, 1] + g_ref[1, 1]


def _tc_final_body(dp, out):
  out[...] = SCALE * (dp[0, 0] + dp[1, 0])


def kernel(fluidPosition, fluidVolume, fluidDistances, fluidRadialDistances,
           fluidDensity, i, j):
  del fluidPosition
  i = i.astype(jnp.int32)
  j = j.astype(jnp.int32)
  pad = EP - E
  iE = jnp.pad(i, (0, pad)).reshape(RROWS, 128)
  jE = jnp.pad(j, (0, pad)).reshape(RROWS, 128)
  dxE = jnp.pad(fluidDistances[:, 0], (0, pad)).reshape(RROWS, 128)
  dyE = jnp.pad(fluidDistances[:, 1], (0, pad)).reshape(RROWS, 128)
  radE = jnp.pad(fluidRadialDistances, (0, pad)).reshape(RROWS, 128)
  volP = jnp.pad(fluidVolume, (0, NP - N))
  densP = jnp.pad(fluidDensity, (0, NP - N))
  zN = jnp.zeros((NP,), jnp.float32)
  f32 = jnp.float32
  shp = jax.ShapeDtypeStruct((NR, NL), f32)

  Mpart = _sc_pass_a(iE, jE, dxE, dyE, radE, volP, zN)

  li0, li1, li2, li3, rho2 = pl.pallas_call(
      _tc_pinv_body, out_shape=[shp] * 5)(
          Mpart.reshape(2, 4, NR, NL), densP.reshape(NR, NL))

  Gpart = _sc_pass_c(iE, jE, dxE, dyE, radE,
                     li0.reshape(NP), li1.reshape(NP), li2.reshape(NP),
                     li3.reshape(NP), rho2.reshape(NP), volP, zN)

  gx2, gy2 = pl.pallas_call(_tc_combine2_body, out_shape=[shp] * 2)(
      Gpart.reshape(2, 2, NR, NL))

  Dpart = _sc_pass_e(iE, jE, dxE, dyE, radE,
                     gx2.reshape(NP), gy2.reshape(NP), rho2.reshape(NP),
                     volP, zN)

  out2 = pl.pallas_call(_tc_final_body, out_shape=shp)(
      Dpart.reshape(2, 1, NR, NL))
  return out2.reshape(NP)[:N]
